# Initial kernel scaffold; baseline (speedup 1.0000x reference)
import jax
import jax.numpy as jnp
from jax.experimental import pallas as pl


def kernel(x, edge_index, W1, b1, W2, b2):
    raise NotImplementedError("write your pallas kernel here")


# speedup vs baseline: 4.7151x; 4.7151x over previous
import functools

import jax
import jax.numpy as jnp
from jax import lax
from jax.experimental import pallas as pl
from jax.experimental.pallas import tpu as pltpu
from jax.experimental.pallas import tpu_sc as plsc

NC = 2
NS = 16
NW = NC * NS
CHUNK = 128


def _sc_agg_call(N, D, E):
    ew = -(-E // NW)
    chunks = -(-ew // CHUNK)
    n_pad = -(-(N + 1) // (NS * 8)) * (NS * 8)
    rpt = n_pad // NS

    def body(h_hbm, src_hbm, dst_hbm, z_hbm, out_hbm, src_v, dst_v, rows_v,
             acc_sh, sem):
        c = lax.axis_index("c")
        s = lax.axis_index("s")
        pltpu.sync_copy(z_hbm, acc_sh.at[pl.ds(s * rpt, rpt)])
        pltpu.sync_copy(src_hbm.at[c, s], src_v)
        pltpu.sync_copy(dst_hbm.at[c, s], dst_v)
        plsc.subcore_barrier()

        def step(j, carry):
            pltpu.async_copy(h_hbm.at[src_v.at[j]], rows_v, sem).wait()
            pltpu.sync_copy(rows_v, acc_sh.at[dst_v.at[j]], add=True)
            return carry

        lax.fori_loop(0, chunks, step, 0)
        plsc.subcore_barrier()
        pltpu.sync_copy(acc_sh.at[pl.ds(s * rpt, rpt)],
                        out_hbm.at[c, pl.ds(s * rpt, rpt)])

    mesh = plsc.VectorSubcoreMesh(core_axis_name="c", subcore_axis_name="s",
                                  num_cores=NC, num_subcores=NS)
    return pl.kernel(
        body,
        out_type=jax.ShapeDtypeStruct((NC, n_pad, D), jnp.float32),
        mesh=mesh,
        scratch_types=[
            pltpu.VMEM((chunks, CHUNK), jnp.int32),
            pltpu.VMEM((chunks, CHUNK), jnp.int32),
            pltpu.VMEM((CHUNK, D), jnp.float32),
            pltpu.VMEM_SHARED((n_pad, D), jnp.float32),
            pltpu.SemaphoreType.DMA,
        ],
    ), chunks, n_pad, rpt


def _mlp_body(relu_out, x_ref, p_ref, w1_ref, b1_ref, w2_ref, b2_ref, o_ref):
    u = x_ref[...] + p_ref[0] + p_ref[1]
    t = jnp.maximum(
        jnp.dot(u, w1_ref[...], preferred_element_type=jnp.float32) + b1_ref[...],
        0.0)
    y = jnp.dot(t, w2_ref[...], preferred_element_type=jnp.float32) + b2_ref[...]
    if relu_out:
        y = jnp.maximum(y, 0.0)
    o_ref[...] = y


def _mlp_call(N, D, n_pad, relu_out, bn=1000):
    return pl.pallas_call(
        functools.partial(_mlp_body, relu_out),
        grid=(N // bn,),
        in_specs=[
            pl.BlockSpec((bn, D), lambda i: (i, 0)),
            pl.BlockSpec((NC, bn, D), lambda i: (0, i, 0)),
            pl.BlockSpec((D, D), lambda i: (0, 0)),
            pl.BlockSpec((1, D), lambda i: (0, 0)),
            pl.BlockSpec((D, D), lambda i: (0, 0)),
            pl.BlockSpec((1, D), lambda i: (0, 0)),
        ],
        out_specs=pl.BlockSpec((bn, D), lambda i: (i, 0)),
        out_shape=jax.ShapeDtypeStruct((N, D), jnp.float32),
    )


def kernel(x, edge_index, W1, b1, W2, b2):
    N, D = x.shape
    E = edge_index.shape[1]

    sc_call, chunks, n_pad, rpt = _sc_agg_call(N, D, E)
    e_pad = NW * chunks * CHUNK

    src = edge_index[0].astype(jnp.int32)
    dst = edge_index[1].astype(jnp.int32)
    pad = e_pad - E
    src_p = jnp.concatenate([src, jnp.zeros((pad,), jnp.int32)])
    dst_p = jnp.concatenate([dst, jnp.full((pad,), N, jnp.int32)])
    src_p = src_p.reshape(NC, NS, chunks, CHUNK)
    dst_p = dst_p.reshape(NC, NS, chunks, CHUNK)
    z_init = jnp.zeros((rpt, D), jnp.float32)

    b1r = b1.reshape(1, D)
    b2r = b2.reshape(1, D)

    h = x
    for layer in range(3):
        partials = sc_call(h, src_p, dst_p, z_init)
        h = _mlp_call(N, D, n_pad, relu_out=(layer < 2))(
            h, partials, W1, b1r, W2, b2r)
    return h
